# baseline (speedup 1.0000x reference)
import functools

import jax
import jax.numpy as jnp
from jax.experimental import pallas as pl
from jax.experimental.pallas import tpu as pltpu

_TOPK = 8
_BLK = 4096
_IMAX = 2**31 - 1


def _fused_topk_kernel(se_ref, q_ref, e_ref, vals_ref, idxs_ref,
                       topv, topi, sc, bm, qn_s, *, k_total, blk, nq):
    step = pl.program_id(0)
    nsteps = pl.num_programs(0)

    @pl.when(step == 0)
    def _init():
        topv[...] = jnp.full((nq, _TOPK), -jnp.inf, dtype=jnp.float32)
        topi[...] = jnp.full((nq, _TOPK), -1, dtype=jnp.int32)
        q = q_ref[...]
        l1 = jnp.maximum(jnp.sum(jnp.abs(q), axis=1, keepdims=True), 1e-12)
        qn_s[...] = q / l1

    scores = jnp.dot(qn_s[...], e_ref[...], preferred_element_type=jnp.float32)

    col = step * blk + jax.lax.broadcasted_iota(jnp.int32, (nq, blk), 1)

    bstart = step * blk
    start = se_ref[0]
    end = se_ref[1]
    needs_mask = ((bstart < end) & (start < bstart + blk)) | (bstart + blk > k_total)

    @pl.when(needs_mask)
    def _masked():
        bad = ((col >= start) & (col < end)) | (col >= k_total)
        s = jnp.where(bad, -jnp.inf, scores)
        sc[...] = s
        bm[...] = jnp.max(s, axis=1, keepdims=True)

    @pl.when(jnp.logical_not(needs_mask))
    def _unmasked():
        bm[...] = jnp.max(scores, axis=1, keepdims=True)

    cond0 = jnp.any(bm[...] > topv[...][:, _TOPK - 1:])

    @pl.when(cond0 & jnp.logical_not(needs_mask))
    def _stash():
        sc[...] = scores

    j8 = jax.lax.broadcasted_iota(jnp.int32, (nq, _TOPK), 1)

    def _extract(_):
        s = sc[...]
        m = bm[...]
        eq = s == m
        mi = jnp.min(jnp.where(eq, col, _IMAX), axis=1, keepdims=True)
        ns = jnp.where(col == mi, -jnp.inf, s)
        sc[...] = ns
        bm[...] = jnp.max(ns, axis=1, keepdims=True)
        tv = topv[...]
        ti = topi[...]
        pos = jnp.sum((tv >= m).astype(jnp.int32), axis=1, keepdims=True)
        tvs = jnp.concatenate([tv[:, :1], tv[:, :_TOPK - 1]], axis=1)
        tis = jnp.concatenate([ti[:, :1], ti[:, :_TOPK - 1]], axis=1)
        topv[...] = jnp.where(j8 < pos, tv, jnp.where(j8 == pos, m, tvs))
        topi[...] = jnp.where(j8 < pos, ti, jnp.where(j8 == pos, mi, tis))
        return jnp.any(bm[...] > topv[...][:, _TOPK - 1:])

    jax.lax.while_loop(lambda c: c, _extract, cond0)

    @pl.when(step == nsteps - 1)
    def _emit():
        vals_ref[...] = topv[...]
        idxs_ref[...] = topi[...]


def kernel(query_emb, embeddings, start, end):
    nq, d = query_emb.shape
    k_total = embeddings.shape[1]
    nsteps = pl.cdiv(k_total, _BLK)
    se = jnp.stack([jnp.asarray(start, jnp.int32), jnp.asarray(end, jnp.int32)])
    grid_spec = pltpu.PrefetchScalarGridSpec(
        num_scalar_prefetch=1,
        grid=(nsteps,),
        in_specs=[
            pl.BlockSpec((nq, d), lambda i, se_ref: (0, 0)),
            pl.BlockSpec((d, _BLK), lambda i, se_ref: (0, i)),
        ],
        out_specs=[
            pl.BlockSpec((nq, _TOPK), lambda i, se_ref: (0, 0)),
            pl.BlockSpec((nq, _TOPK), lambda i, se_ref: (0, 0)),
        ],
        scratch_shapes=[
            pltpu.VMEM((nq, _TOPK), jnp.float32),
            pltpu.VMEM((nq, _TOPK), jnp.int32),
            pltpu.VMEM((nq, _BLK), jnp.float32),
            pltpu.VMEM((nq, 1), jnp.float32),
            pltpu.VMEM((nq, d), jnp.float32),
        ],
    )
    vals, idxs = pl.pallas_call(
        functools.partial(_fused_topk_kernel, k_total=k_total, blk=_BLK, nq=nq),
        grid_spec=grid_spec,
        out_shape=[
            jax.ShapeDtypeStruct((nq, _TOPK), jnp.float32),
            jax.ShapeDtypeStruct((nq, _TOPK), jnp.int32),
        ],
        compiler_params=pltpu.CompilerParams(dimension_semantics=("arbitrary",)),
    )(se, query_emb, embeddings)
    return vals, idxs


# speedup vs baseline: 1.2209x; 1.2209x over previous
import functools

import jax
import jax.numpy as jnp
from jax.experimental import pallas as pl
from jax.experimental.pallas import tpu as pltpu

_TOPK = 8
_BLK = 8192
_IMAX = 2**31 - 1


def _fused_topk_kernel(se_ref, q_ref, e_ref, vals_ref, idxs_ref,
                       topv, topi, sc, bm, qn_s, *, k_total, blk, nq):
    step = pl.program_id(0)
    nsteps = pl.num_programs(0)

    @pl.when(step == 0)
    def _init():
        topv[...] = jnp.full((nq, _TOPK), -jnp.inf, dtype=jnp.float32)
        topi[...] = jnp.full((nq, _TOPK), -1, dtype=jnp.int32)
        q = q_ref[...]
        l1 = jnp.maximum(jnp.sum(jnp.abs(q), axis=1, keepdims=True), 1e-12)
        qn_s[...] = q / l1

    scores = jnp.dot(qn_s[...], e_ref[...], preferred_element_type=jnp.float32)

    col = step * blk + jax.lax.broadcasted_iota(jnp.int32, (nq, blk), 1)

    bstart = step * blk
    start = se_ref[0]
    end = se_ref[1]
    needs_mask = ((bstart < end) & (start < bstart + blk)) | (bstart + blk > k_total)

    @pl.when(needs_mask)
    def _masked():
        bad = ((col >= start) & (col < end)) | (col >= k_total)
        s = jnp.where(bad, -jnp.inf, scores)
        sc[...] = s
        bm[...] = jnp.max(s, axis=1, keepdims=True)

    @pl.when(jnp.logical_not(needs_mask))
    def _unmasked():
        bm[...] = jnp.max(scores, axis=1, keepdims=True)

    cond0 = jnp.any(bm[...] > topv[...][:, _TOPK - 1:])

    @pl.when(cond0 & jnp.logical_not(needs_mask))
    def _stash():
        sc[...] = scores

    j8 = jax.lax.broadcasted_iota(jnp.int32, (nq, _TOPK), 1)

    def _extract(_):
        s = sc[...]
        m = bm[...]
        eq = s == m
        mi = jnp.min(jnp.where(eq, col, _IMAX), axis=1, keepdims=True)
        ns = jnp.where(col == mi, -jnp.inf, s)
        sc[...] = ns
        bm[...] = jnp.max(ns, axis=1, keepdims=True)
        tv = topv[...]
        ti = topi[...]
        pos = jnp.sum((tv >= m).astype(jnp.int32), axis=1, keepdims=True)
        tvs = jnp.concatenate([tv[:, :1], tv[:, :_TOPK - 1]], axis=1)
        tis = jnp.concatenate([ti[:, :1], ti[:, :_TOPK - 1]], axis=1)
        topv[...] = jnp.where(j8 < pos, tv, jnp.where(j8 == pos, m, tvs))
        topi[...] = jnp.where(j8 < pos, ti, jnp.where(j8 == pos, mi, tis))
        return jnp.any(bm[...] > topv[...][:, _TOPK - 1:])

    jax.lax.while_loop(lambda c: c, _extract, cond0)

    @pl.when(step == nsteps - 1)
    def _emit():
        vals_ref[...] = topv[...]
        idxs_ref[...] = topi[...]


def kernel(query_emb, embeddings, start, end):
    nq, d = query_emb.shape
    k_total = embeddings.shape[1]
    nsteps = pl.cdiv(k_total, _BLK)
    se = jnp.stack([jnp.asarray(start, jnp.int32), jnp.asarray(end, jnp.int32)])
    grid_spec = pltpu.PrefetchScalarGridSpec(
        num_scalar_prefetch=1,
        grid=(nsteps,),
        in_specs=[
            pl.BlockSpec((nq, d), lambda i, se_ref: (0, 0)),
            pl.BlockSpec((d, _BLK), lambda i, se_ref: (0, i)),
        ],
        out_specs=[
            pl.BlockSpec((nq, _TOPK), lambda i, se_ref: (0, 0)),
            pl.BlockSpec((nq, _TOPK), lambda i, se_ref: (0, 0)),
        ],
        scratch_shapes=[
            pltpu.VMEM((nq, _TOPK), jnp.float32),
            pltpu.VMEM((nq, _TOPK), jnp.int32),
            pltpu.VMEM((nq, _BLK), jnp.float32),
            pltpu.VMEM((nq, 1), jnp.float32),
            pltpu.VMEM((nq, d), jnp.float32),
        ],
    )
    vals, idxs = pl.pallas_call(
        functools.partial(_fused_topk_kernel, k_total=k_total, blk=_BLK, nq=nq),
        grid_spec=grid_spec,
        out_shape=[
            jax.ShapeDtypeStruct((nq, _TOPK), jnp.float32),
            jax.ShapeDtypeStruct((nq, _TOPK), jnp.int32),
        ],
        compiler_params=pltpu.CompilerParams(dimension_semantics=("arbitrary",)),
    )(se, query_emb, embeddings)
    return vals, idxs
